# Initial kernel scaffold; baseline (speedup 1.0000x reference)
import jax
import jax.numpy as jnp
from jax.experimental import pallas as pl


def kernel(x, W_r, b_r, W_e, b_e, W_q, b_q, W_a, b_a):
    raise NotImplementedError("write your pallas kernel here")


# speedup vs baseline: 1.5444x; 1.5444x over previous
import functools

import jax
import jax.numpy as jnp
from jax.experimental import pallas as pl
from jax.experimental.pallas import tpu as pltpu

N_TOK = 2048
D_IN = 1024
HID = 1024
D_OUT = 1024
NUM_CHUNKS = 8
TOP_K = 2
THRESHOLD = 0.2

BN = 256


def _router_body(x_ref, wr_ref, br_ref, wq_ref,
                 gated_ref, act_ref, ma_ref, wb_ref):
    x = x_ref[...]
    logits = jnp.dot(x, wr_ref[...], preferred_element_type=jnp.float32)
    logits = logits + br_ref[...]
    m = jnp.max(logits, axis=-1, keepdims=True)
    e = jnp.exp(logits - m)
    gates = e / jnp.sum(e, axis=-1, keepdims=True)

    lane = jax.lax.broadcasted_iota(jnp.int32, gates.shape, 1)
    g1 = jnp.max(gates, axis=-1, keepdims=True)
    i1 = jnp.min(jnp.where(gates >= g1, lane, NUM_CHUNKS), axis=-1,
                 keepdims=True)
    mask1 = lane == i1
    masked = jnp.where(mask1, -jnp.inf, gates)
    g2 = jnp.max(masked, axis=-1, keepdims=True)
    i2 = jnp.min(jnp.where(masked >= g2, lane, NUM_CHUNKS), axis=-1,
                 keepdims=True)
    mask = mask1 | (lane == i2)
    gated = jnp.where(mask, gates, 0.0)
    gated_ref[...] = gated

    acts = jnp.sum(gated, axis=0, keepdims=True) * (1.0 / N_TOK)
    act_ref[...] = acts
    ma = jnp.sum(acts) * (1.0 / NUM_CHUNKS)
    ma_ref[...] = jnp.full((1, 1), ma, dtype=jnp.float32)

    wq = wq_ref[...]
    scale = jnp.max(jnp.abs(wq)) * (1.0 / 127.0)
    wfq = jnp.round(wq / scale) * scale
    wb_ref[...] = ma * wq + (1.0 - ma) * wfq


def _expert_body(x_ref, gated_ref, we_ref, be_ref, wb_ref, bq_ref,
                 x3_ref, asum_ref):
    t = pl.program_id(0)
    x = x_ref[...]
    gated = gated_ref[...]
    acc = jnp.zeros((x.shape[0], HID), dtype=jnp.float32)
    for c in range(NUM_CHUNKS):
        g = gated[:, c:c + 1]
        acc = acc + g * (jnp.dot(x, we_ref[c], preferred_element_type=jnp.float32)
                         + be_ref[c])
    x3 = jnp.dot(acc, wb_ref[...], preferred_element_type=jnp.float32)
    x3 = x3 + bq_ref[...]
    x3_ref[...] = x3
    psum = jnp.full((1, 1), jnp.sum(jnp.abs(x3)), dtype=jnp.float32)

    @pl.when(t == 0)
    def _():
        asum_ref[...] = jnp.zeros((1, 1), dtype=jnp.float32)

    asum_ref[...] += psum


def _final_body(x3_ref, wa_ref, ba_ref, asum_ref, out_ref):
    act = asum_ref[...] * (1.0 / (N_TOK * HID))
    ind = jnp.where(act > THRESHOLD, 1.0, 0.0)
    out = jnp.dot(x3_ref[...], wa_ref[...], preferred_element_type=jnp.float32)
    out_ref[...] = (out + ba_ref[...]) * ind


@jax.jit
def _run(x, W_r, b_r, W_e, b_e, W_q, b_q, W_a, b_a):
    f32 = jnp.float32
    gated, acts, ma, W_blend = pl.pallas_call(
        _router_body,
        out_shape=(
            jax.ShapeDtypeStruct((N_TOK, NUM_CHUNKS), f32),
            jax.ShapeDtypeStruct((1, NUM_CHUNKS), f32),
            jax.ShapeDtypeStruct((1, 1), f32),
            jax.ShapeDtypeStruct((HID, HID), f32),
        ),
    )(x, W_r, b_r.reshape(1, NUM_CHUNKS), W_q)

    nt = N_TOK // BN
    x3, asum = pl.pallas_call(
        _expert_body,
        grid=(nt,),
        in_specs=[
            pl.BlockSpec((BN, D_IN), lambda t: (t, 0)),
            pl.BlockSpec((BN, NUM_CHUNKS), lambda t: (t, 0)),
            pl.BlockSpec((NUM_CHUNKS, D_IN, HID), lambda t: (0, 0, 0)),
            pl.BlockSpec((NUM_CHUNKS, HID), lambda t: (0, 0)),
            pl.BlockSpec((HID, HID), lambda t: (0, 0)),
            pl.BlockSpec((1, HID), lambda t: (0, 0)),
        ],
        out_specs=(
            pl.BlockSpec((BN, HID), lambda t: (t, 0)),
            pl.BlockSpec((1, 1), lambda t: (0, 0)),
        ),
        out_shape=(
            jax.ShapeDtypeStruct((N_TOK, HID), f32),
            jax.ShapeDtypeStruct((1, 1), f32),
        ),
        compiler_params=pltpu.CompilerParams(
            dimension_semantics=("arbitrary",),
        ),
    )(x, gated, W_e, b_e, W_blend, b_q.reshape(1, HID))

    out = pl.pallas_call(
        _final_body,
        grid=(nt,),
        in_specs=[
            pl.BlockSpec((BN, HID), lambda t: (t, 0)),
            pl.BlockSpec((HID, D_OUT), lambda t: (0, 0)),
            pl.BlockSpec((1, D_OUT), lambda t: (0, 0)),
            pl.BlockSpec((1, 1), lambda t: (0, 0)),
        ],
        out_specs=pl.BlockSpec((BN, D_OUT), lambda t: (t, 0)),
        out_shape=jax.ShapeDtypeStruct((N_TOK, D_OUT), f32),
    )(x3, W_a, b_a.reshape(1, D_OUT), asum)

    act = asum[0, 0] * (1.0 / (N_TOK * HID))
    return out, acts.reshape(NUM_CHUNKS), ma.reshape(()), act


def kernel(x, W_r, b_r, W_e, b_e, W_q, b_q, W_a, b_a):
    return _run(x, W_r, b_r, W_e, b_e, W_q, b_q, W_a, b_a)
